# baseline (speedup 1.0000x reference)
import functools

import jax
import jax.numpy as jnp
from jax import lax
from jax.experimental import pallas as pl
from jax.experimental.pallas import tpu as pltpu
from jax.experimental.pallas import tpu_sc as plsc

B = 16384
F = 26
V = 100000
D = 32
NUM = 13

NC = 2
NS = 16
NW = NC * NS

ROWS = F * D
ROWS_PER_W = ROWS // NW
OCHUNK = 4096
N_OCHUNK = B // OCHUNK


def _gather_kernel(tab_hbm, xt_hbm, out_hbm, row_v, idx_v, out_v, sems):
    wid = lax.axis_index("s") * NC + lax.axis_index("c")
    base_r = wid * ROWS_PER_W

    def row_body(j, _):
        r = base_r + j
        f = r >> 5

        stage = pltpu.async_copy(tab_hbm.at[r], row_v, sems.at[2])

        @pl.when((j == 0) | ((r & 31) == 0))
        def _load_idx():
            pltpu.sync_copy(xt_hbm.at[f], idx_v)

        stage.wait()

        def chunk_body(c, _):
            cbase = c * OCHUNK
            slot = lax.rem(c, 2)
            dst = out_hbm.at[r, pl.ds(c * (OCHUNK // 2), OCHUNK // 2)]

            @pl.when(c >= 2)
            def _drain():
                pltpu.make_async_copy(out_v.at[slot], dst, sems.at[slot]).wait()

            @plsc.parallel_loop(0, OCHUNK // 32, unroll=8)
            def vec_body(i):
                ids_a = idx_v[pl.ds(cbase + i * 32, 16)]
                ids_b = idx_v[pl.ds(cbase + i * 32 + 16, 16)]
                va = plsc.load_gather(row_v, [ids_a])
                vb = plsc.load_gather(row_v, [ids_b])
                packed = plsc.pack(va, vb, format=plsc.PackFormat.INTERLEAVED)
                out_v[slot, pl.ds(i * 16, 16)] = plsc.bitcast(packed, jnp.int32)

            pltpu.async_copy(out_v.at[slot], dst, sems.at[slot])
            return 0

        lax.fori_loop(0, N_OCHUNK, chunk_body, 0)
        for slot in range(2):
            pltpu.make_async_copy(
                out_v.at[slot], out_hbm.at[r, pl.ds(0, OCHUNK // 2)],
                sems.at[slot]
            ).wait()
        return 0

    lax.fori_loop(0, ROWS_PER_W, row_body, 0)


@functools.cache
def _sc_gather():
    return pl.kernel(
        _gather_kernel,
        out_type=jax.ShapeDtypeStruct((ROWS, B // 2), jnp.int32),
        mesh=plsc.VectorSubcoreMesh(core_axis_name="c", subcore_axis_name="s"),
        scratch_types=[
            pltpu.VMEM((V,), jnp.float32),
            pltpu.VMEM((B,), jnp.int32),
            pltpu.VMEM((2, OCHUNK // 2), jnp.int32),
            pltpu.SemaphoreType.DMA((3,)),
        ],
        compiler_params=pltpu.CompilerParams(
            use_tc_tiling_on_sc=True,
            needs_layout_passes=False,
        ),
    )


BB = 512


def _tower(h, xn, w1_ref, b1_ref, w2_ref, b2_ref, w3_ref, b3_ref,
           wpd_ref, wpw_ref, bp_ref):
    h1 = lax.dot_general(h, w1_ref[...], (((0,), (0,)), ((), ())),
                         preferred_element_type=jnp.float32)
    h1 = h1 + b1_ref[...]
    h1 = jnp.where(h1 > 0, h1, 0.01 * h1)
    h2 = jnp.dot(h1, w2_ref[...], preferred_element_type=jnp.float32)
    h2 = h2 + b2_ref[...]
    h2 = jnp.where(h2 > 0, h2, 0.01 * h2)
    h3 = jnp.dot(h2, w3_ref[...], preferred_element_type=jnp.float32)
    h3 = jnp.tanh(h3 + b3_ref[...])
    z = (jnp.dot(h3, wpd_ref[...], preferred_element_type=jnp.float32)
         + jnp.dot(xn, wpw_ref[...], preferred_element_type=jnp.float32)
         + bp_ref[...])
    return 1.0 / (1.0 + jnp.exp(-z))


def _mlp_kernel(hp_ref, xne_ref, xno_ref, w1_ref, b1_ref, w2_ref, b2_ref,
                w3_ref, b3_ref, wpd_ref, wpw_ref, bp_ref, oe_ref, oo_ref):
    u = hp_ref[...]
    he = lax.bitcast_convert_type(
        lax.shift_left(u, 16), jnp.float32).astype(jnp.bfloat16)
    ho = lax.bitcast_convert_type(
        lax.bitwise_and(u, jnp.int32(-65536)),
        jnp.float32).astype(jnp.bfloat16)
    args = (w1_ref, b1_ref, w2_ref, b2_ref, w3_ref, b3_ref,
            wpd_ref, wpw_ref, bp_ref)
    oe_ref[...] = _tower(he, xne_ref[...], *args)
    oo_ref[...] = _tower(ho, xno_ref[...], *args)


def _mlp(hp, x_numerical, W1, b1, W2, b2, W3, b3, Wp, bp):
    wpd = Wp[:64]
    wpw = Wp[64:]
    xn3 = x_numerical.reshape(B // 32, 32, NUM)
    xne = xn3[:, :16].reshape(B // 2, NUM)
    xno = xn3[:, 16:].reshape(B // 2, NUM)
    grid = (B // BB,)
    fixed = lambda i: (0, 0)
    hb = BB // 2
    oe, oo = pl.pallas_call(
        _mlp_kernel,
        grid=grid,
        in_specs=[
            pl.BlockSpec((F * D, hb), lambda i: (0, i)),
            pl.BlockSpec((hb, NUM), lambda i: (i, 0)),
            pl.BlockSpec((hb, NUM), lambda i: (i, 0)),
            pl.BlockSpec((F * D, 256), fixed),
            pl.BlockSpec((1, 256), fixed),
            pl.BlockSpec((256, 128), fixed),
            pl.BlockSpec((1, 128), fixed),
            pl.BlockSpec((128, 64), fixed),
            pl.BlockSpec((1, 64), fixed),
            pl.BlockSpec((64, 1), fixed),
            pl.BlockSpec((NUM, 1), fixed),
            pl.BlockSpec((1, 1), fixed),
        ],
        out_specs=[
            pl.BlockSpec((hb, 1), lambda i: (i, 0)),
            pl.BlockSpec((hb, 1), lambda i: (i, 0)),
        ],
        out_shape=[
            jax.ShapeDtypeStruct((B // 2, 1), jnp.float32),
            jax.ShapeDtypeStruct((B // 2, 1), jnp.float32),
        ],
    )(hp, xne, xno, W1.astype(jnp.bfloat16), b1.reshape(1, 256),
      W2, b2.reshape(1, 128),
      W3, b3.reshape(1, 64), wpd, wpw, bp.reshape(1, 1))
    return jnp.concatenate(
        [oe.reshape(B // 32, 16), oo.reshape(B // 32, 16)], axis=1
    ).reshape(B, 1)


def kernel(x_numerical, x_categorical, tables, W1, b1, W2, b2, W3, b3, Wp, bp):
    tabt = tables.transpose(0, 2, 1).reshape(ROWS, V)
    xt = x_categorical.T
    hp = _sc_gather()(tabt, xt)
    return _mlp(hp, x_numerical, W1, b1, W2, b2, W3, b3, Wp, bp)


# speedup vs baseline: 1.0617x; 1.0617x over previous
import functools

import jax
import jax.numpy as jnp
from jax import lax
from jax.experimental import pallas as pl
from jax.experimental.pallas import tpu as pltpu
from jax.experimental.pallas import tpu_sc as plsc

B = 16384
F = 26
V = 100000
D = 32
NUM = 13

NC = 2
NS = 16
NW = NC * NS

ROWS = F * D
ROWS_PER_W = ROWS // NW
OCHUNK = 4096
N_OCHUNK = B // OCHUNK


def _gather_kernel(tab_hbm, xt_hbm, out_hbm, row_v, idx_v, out_v, sems):
    wid = lax.axis_index("s") * NC + lax.axis_index("c")
    base_r = wid * ROWS_PER_W

    def row_body(j, _):
        r = base_r + j
        f = r >> 5

        stage = pltpu.async_copy(tab_hbm.at[r], row_v, sems.at[2])

        @pl.when((j == 0) | ((r & 31) == 0))
        def _load_idx():
            pltpu.sync_copy(xt_hbm.at[f], idx_v)

        stage.wait()

        def chunk_body(c, _):
            cbase = c * OCHUNK
            slot = lax.rem(c, 2)
            dst = out_hbm.at[r, pl.ds(c * (OCHUNK // 2), OCHUNK // 2)]

            @pl.when(c >= 2)
            def _drain():
                pltpu.make_async_copy(out_v.at[slot], dst, sems.at[slot]).wait()

            @plsc.parallel_loop(0, OCHUNK // 32, unroll=8)
            def vec_body(i):
                ids_a = idx_v[pl.ds(cbase + i * 32, 16)]
                ids_b = idx_v[pl.ds(cbase + i * 32 + 16, 16)]
                va = plsc.load_gather(row_v, [ids_a])
                vb = plsc.load_gather(row_v, [ids_b])
                packed = plsc.pack(va, vb, format=plsc.PackFormat.INTERLEAVED)
                out_v[slot, pl.ds(i * 16, 16)] = plsc.bitcast(packed, jnp.int32)

            pltpu.async_copy(out_v.at[slot], dst, sems.at[slot])
            return 0

        lax.fori_loop(0, N_OCHUNK, chunk_body, 0)
        for slot in range(2):
            pltpu.make_async_copy(
                out_v.at[slot], out_hbm.at[r, pl.ds(0, OCHUNK // 2)],
                sems.at[slot]
            ).wait()
        return 0

    lax.fori_loop(0, ROWS_PER_W, row_body, 0)


@functools.cache
def _sc_gather():
    return pl.kernel(
        _gather_kernel,
        out_type=jax.ShapeDtypeStruct((ROWS, B // 2), jnp.int32),
        mesh=plsc.VectorSubcoreMesh(core_axis_name="c", subcore_axis_name="s"),
        scratch_types=[
            pltpu.VMEM((V,), jnp.float32),
            pltpu.VMEM((B,), jnp.int32),
            pltpu.VMEM((2, OCHUNK // 2), jnp.int32),
            pltpu.SemaphoreType.DMA((3,)),
        ],
        compiler_params=pltpu.CompilerParams(
            use_tc_tiling_on_sc=True,
            needs_layout_passes=False,
        ),
    )


BB = 1024


def _tower(h, xn, w1_ref, b1_ref, w2_ref, b2_ref, w3_ref, b3_ref,
           wpd_ref, wpw_ref, bp_ref):
    h1 = lax.dot_general(h, w1_ref[...], (((0,), (0,)), ((), ())),
                         preferred_element_type=jnp.float32)
    h1 = h1 + b1_ref[...]
    h1 = jnp.where(h1 > 0, h1, 0.01 * h1)
    h2 = jnp.dot(h1, w2_ref[...], preferred_element_type=jnp.float32)
    h2 = h2 + b2_ref[...]
    h2 = jnp.where(h2 > 0, h2, 0.01 * h2)
    h3 = jnp.dot(h2, w3_ref[...], preferred_element_type=jnp.float32)
    h3 = jnp.tanh(h3 + b3_ref[...])
    z = (jnp.dot(h3, wpd_ref[...], preferred_element_type=jnp.float32)
         + jnp.dot(xn, wpw_ref[...], preferred_element_type=jnp.float32)
         + bp_ref[...])
    return 1.0 / (1.0 + jnp.exp(-z))


def _mlp_kernel(hp_ref, xne_ref, xno_ref, w1_ref, b1_ref, w2_ref, b2_ref,
                w3_ref, b3_ref, wpd_ref, wpw_ref, bp_ref, oe_ref, oo_ref):
    u = hp_ref[...]
    he = lax.bitcast_convert_type(
        lax.shift_left(u, 16), jnp.float32).astype(jnp.bfloat16)
    ho = lax.bitcast_convert_type(
        lax.bitwise_and(u, jnp.int32(-65536)),
        jnp.float32).astype(jnp.bfloat16)
    args = (w1_ref, b1_ref, w2_ref, b2_ref, w3_ref, b3_ref,
            wpd_ref, wpw_ref, bp_ref)
    oe_ref[...] = _tower(he, xne_ref[...], *args)
    oo_ref[...] = _tower(ho, xno_ref[...], *args)


def _mlp(hp, x_numerical, W1, b1, W2, b2, W3, b3, Wp, bp):
    wpd = Wp[:64]
    wpw = Wp[64:]
    xn3 = x_numerical.reshape(B // 32, 32, NUM)
    xne = xn3[:, :16].reshape(B // 2, NUM)
    xno = xn3[:, 16:].reshape(B // 2, NUM)
    grid = (B // BB,)
    fixed = lambda i: (0, 0)
    hb = BB // 2
    oe, oo = pl.pallas_call(
        _mlp_kernel,
        grid=grid,
        in_specs=[
            pl.BlockSpec((F * D, hb), lambda i: (0, i)),
            pl.BlockSpec((hb, NUM), lambda i: (i, 0)),
            pl.BlockSpec((hb, NUM), lambda i: (i, 0)),
            pl.BlockSpec((F * D, 256), fixed),
            pl.BlockSpec((1, 256), fixed),
            pl.BlockSpec((256, 128), fixed),
            pl.BlockSpec((1, 128), fixed),
            pl.BlockSpec((128, 64), fixed),
            pl.BlockSpec((1, 64), fixed),
            pl.BlockSpec((64, 1), fixed),
            pl.BlockSpec((NUM, 1), fixed),
            pl.BlockSpec((1, 1), fixed),
        ],
        out_specs=[
            pl.BlockSpec((hb, 1), lambda i: (i, 0)),
            pl.BlockSpec((hb, 1), lambda i: (i, 0)),
        ],
        out_shape=[
            jax.ShapeDtypeStruct((B // 2, 1), jnp.float32),
            jax.ShapeDtypeStruct((B // 2, 1), jnp.float32),
        ],
    )(hp, xne, xno, W1.astype(jnp.bfloat16), b1.reshape(1, 256),
      W2, b2.reshape(1, 128),
      W3, b3.reshape(1, 64), wpd, wpw, bp.reshape(1, 1))
    return jnp.concatenate(
        [oe.reshape(B // 32, 16), oo.reshape(B // 32, 16)], axis=1
    ).reshape(B, 1)


def kernel(x_numerical, x_categorical, tables, W1, b1, W2, b2, W3, b3, Wp, bp):
    tabt = tables.transpose(0, 2, 1).reshape(ROWS, V)
    xt = x_categorical.T
    hp = _sc_gather()(tabt, xt)
    return _mlp(hp, x_numerical, W1, b1, W2, b2, W3, b3, Wp, bp)
